# Initial kernel scaffold; baseline (speedup 1.0000x reference)
import jax
import jax.numpy as jnp
from jax.experimental import pallas as pl


def kernel(job_ids, metadata_table, loc_emb, cls_emb, sub_emb, wt_emb):
    raise NotImplementedError("write your pallas kernel here")


# speedup vs baseline: 1.8021x; 1.8021x over previous
import functools

import jax
import jax.numpy as jnp
from jax import lax
from jax.experimental import pallas as pl
from jax.experimental.pallas import tpu as pltpu
from jax.experimental.pallas import tpu_sc as plsc

B = 4096
H = 50
N = B * H
D = 64
NC = 2
NS = 16
NW = NC * NS
PER_W = N // NW
CH = 128
NCHUNK = PER_W // CH


@functools.partial(
    pl.kernel,
    out_type=tuple(jax.ShapeDtypeStruct((N, D), jnp.float32) for _ in range(4)),
    mesh=plsc.VectorSubcoreMesh(core_axis_name="c", subcore_axis_name="s"),
    compiler_params=pltpu.CompilerParams(use_tc_tiling_on_sc=False),
    scratch_types=[
        pltpu.VMEM((CH,), jnp.int32),
        pltpu.VMEM((CH,), jnp.int32),
        pltpu.VMEM((CH, D), jnp.float32),
        pltpu.SemaphoreType.DMA,
    ],
)
def _sc_lookup(jobs, col0, col1, col2, col3, t0, t1, t2, t3,
               o0, o1, o2, o3, idx_v, fidx_v, rows_v, sem):
    wid = lax.axis_index("s") * NC + lax.axis_index("c")
    base = wid * PER_W
    cols = (col0, col1, col2, col3)
    tbls = (t0, t1, t2, t3)
    outs = (o0, o1, o2, o3)

    def chunk_body(j, carry):
        off = base + j * CH
        pltpu.sync_copy(jobs.at[pl.ds(off, CH)], idx_v)
        for f in range(4):
            pltpu.async_copy(cols[f].at[idx_v], fidx_v, sem).wait()
            pltpu.async_copy(tbls[f].at[fidx_v], rows_v, sem).wait()
            pltpu.sync_copy(rows_v, outs[f].at[pl.ds(off, CH)])
        return carry

    lax.fori_loop(0, NCHUNK, chunk_body, 0)


def kernel(job_ids, metadata_table, loc_emb, cls_emb, sub_emb, wt_emb):
    jobs_flat = job_ids.reshape(N).astype(jnp.int32)
    cols = [metadata_table[:, f] for f in range(4)]
    outs = _sc_lookup(jobs_flat, *cols, loc_emb, cls_emb, sub_emb, wt_emb)
    return tuple(o.reshape(B, H, D) for o in outs)


# speedup vs baseline: 1.8042x; 1.0012x over previous
import functools

import jax
import jax.numpy as jnp
from jax import lax
from jax.experimental import pallas as pl
from jax.experimental.pallas import tpu as pltpu
from jax.experimental.pallas import tpu_sc as plsc

B = 4096
H = 50
N = B * H
D = 64
NC = 2
NS = 16
NW = NC * NS
CH = 128
PER_W = N // NW
NCHUNK = PER_W // CH


@functools.partial(
    pl.kernel,
    out_type=tuple(jax.ShapeDtypeStruct((N, D), jnp.float32) for _ in range(4)),
    mesh=plsc.VectorSubcoreMesh(core_axis_name="c", subcore_axis_name="s"),
    compiler_params=pltpu.CompilerParams(use_tc_tiling_on_sc=False),
    scratch_types=[
        pltpu.VMEM((NCHUNK, CH), jnp.int32),
        pltpu.VMEM((4, NCHUNK, CH), jnp.int32),
        pltpu.VMEM((2, 4, CH, D), jnp.float32),
        pltpu.SemaphoreType.DMA((2,)),
        pltpu.SemaphoreType.DMA((2,)),
        pltpu.SemaphoreType.DMA,
    ],
)
def _sc_lookup(jobs, col0, col1, col2, col3, t0, t1, t2, t3,
               o0, o1, o2, o3, idx_v, fid_v, rows_v, sem_f, sem_g, sem_s):
    wid = lax.axis_index("s") * NC + lax.axis_index("c")
    base = wid * PER_W
    cols = (col0, col1, col2, col3)
    tbls = (t0, t1, t2, t3)
    outs = (o0, o1, o2, o3)

    def fid_copies(k):
        slot = lax.rem(k, 2)
        return [pltpu.make_async_copy(cols[f].at[idx_v.at[k]], fid_v.at[f, k],
                                      sem_f.at[slot]) for f in range(4)]

    def fire_fid(k):
        for c in fid_copies(k):
            c.start()

    def wait_fid(k):
        for c in fid_copies(k):
            c.wait()

    def row_copies(k):
        buf = lax.rem(k, 2)
        return [pltpu.make_async_copy(tbls[f].at[fid_v.at[f, k]],
                                      rows_v.at[buf, f], sem_g.at[buf])
                for f in range(4)]

    def fire_rows(k):
        for c in row_copies(k):
            c.start()

    def wait_rows(k):
        for c in row_copies(k):
            c.wait()

    def fire_store(k):
        buf = lax.rem(k, 2)
        for f in range(4):
            pltpu.make_async_copy(rows_v.at[buf, f],
                                  outs[f].at[pl.ds(base + k * CH, CH)],
                                  sem_s).start()

    def wait_store(k):
        buf = lax.rem(k, 2)
        for f in range(4):
            pltpu.make_async_copy(rows_v.at[buf, f],
                                  outs[f].at[pl.ds(base + k * CH, CH)],
                                  sem_s).wait()

    pltpu.sync_copy(jobs.at[wid], idx_v)

    fire_fid(0)
    fire_fid(1)
    wait_fid(0)
    fire_rows(0)

    def chunk_body(k, carry):

        @pl.when(k + 2 < NCHUNK)
        def _():
            fire_fid(k + 2)

        @pl.when(k >= 1)
        def _():
            wait_store(k - 1)

        @pl.when(k + 1 < NCHUNK)
        def _():
            wait_fid(k + 1)
            fire_rows(k + 1)

        wait_rows(k)
        fire_store(k)
        return carry

    lax.fori_loop(0, NCHUNK, chunk_body, 0)
    wait_store(NCHUNK - 1)


def kernel(job_ids, metadata_table, loc_emb, cls_emb, sub_emb, wt_emb):
    jobs = job_ids.reshape(NW, NCHUNK, CH).astype(jnp.int32)
    cols = [metadata_table[:, f] for f in range(4)]
    outs = _sc_lookup(jobs, *cols, loc_emb, cls_emb, sub_emb, wt_emb)
    return tuple(o.reshape(B, H, D) for o in outs)
